# Initial kernel scaffold; baseline (speedup 1.0000x reference)
import jax
import jax.numpy as jnp
from jax.experimental import pallas as pl


def kernel(user_nodes, item_nodes, edge_index, features, preference, W1, b1, W2, b2, Wc1, Wc2):
    raise NotImplementedError("write your pallas kernel here")


# speedup vs baseline: 9.4082x; 9.4082x over previous
import functools

import jax
import jax.numpy as jnp
from jax import lax
from jax.experimental import pallas as pl
from jax.experimental.pallas import tpu as pltpu
from jax.experimental.pallas import tpu_sc as plsc

F32 = jnp.float32

NUSER = 10000
NITEM = 40000
NREAL = 50000
NPAD = 50176
HALF = 32
DL = 64
DF = 128

E = 800000
E2 = 2 * E
EPAD = 1638400
NCHUNK = EPAD // 128
NTILE = 16
CPT = NCHUNK // NTILE
GROUP = 4
ROWS_PT = NPAD // NTILE
ZRC = 98
DEGC = 448

BLK = 400
BLK3 = 512


def _leaky(x):
    return jnp.where(x >= 0, x, 0.01 * x)


def _pref_body(p_ref, wc1_ref, lo_ref, hi_ref):
    p = p_ref[...]
    nrm = jnp.sqrt(jnp.sum(p * p, axis=1, keepdims=True))
    xn = p / jnp.clip(nrm, 1e-12, None)
    y = jnp.dot(xn, wc1_ref[...], preferred_element_type=F32)
    lo_ref[...] = y[:, :HALF]
    hi_ref[...] = y[:, HALF:]


def _item_body(f_ref, w1_ref, b1_ref, w2_ref, b2_ref, wc1_ref, lo_ref, hi_ref):
    f = f_ref[...]
    h = lax.dot_general(f, w1_ref[...], (((1,), (1,)), ((), ())),
                        preferred_element_type=F32) + b1_ref[...][None, :]
    h = _leaky(h)
    h = lax.dot_general(h, w2_ref[...], (((1,), (1,)), ((), ())),
                        preferred_element_type=F32) + b2_ref[...][None, :]
    h = _leaky(h)
    nrm = jnp.sqrt(jnp.sum(h * h, axis=1, keepdims=True))
    xn = h / jnp.clip(nrm, 1e-12, None)
    y = jnp.dot(xn, wc1_ref[...], preferred_element_type=F32)
    lo_ref[...] = y[:, :HALF]
    hi_ref[...] = y[:, HALF:]


def _mid_body(lo_ref, hi_ref, deg_ref, wc2_ref, olo_ref, ohi_ref):
    agg = jnp.concatenate([lo_ref[...], hi_ref[...]], axis=1)
    rdeg = 1.0 / jnp.clip(deg_ref[...], 1.0, None)[:, None]
    x1 = _leaky(agg * rdeg)
    y = jnp.dot(x1, wc2_ref[...], preferred_element_type=F32)
    olo_ref[...] = y[:, :HALF]
    ohi_ref[...] = y[:, HALF:]


def _fin_body(lo_ref, hi_ref, deg_ref, out_ref):
    agg = jnp.concatenate([lo_ref[...], hi_ref[...]], axis=1)
    rdeg = 1.0 / jnp.clip(deg_ref[...], 1.0, None)[:, None]
    out_ref[...] = _leaky(agg * rdeg)


def _run_pref(preference, Wc1):
    return pl.pallas_call(
        _pref_body,
        grid=(NUSER // BLK,),
        in_specs=[
            pl.BlockSpec((BLK, DL), lambda i: (i, 0)),
            pl.BlockSpec((DL, DL), lambda i: (0, 0)),
        ],
        out_specs=[
            pl.BlockSpec((BLK, HALF), lambda i: (i, 0)),
            pl.BlockSpec((BLK, HALF), lambda i: (i, 0)),
        ],
        out_shape=[jax.ShapeDtypeStruct((NUSER, HALF), F32)] * 2,
    )(preference, Wc1)


def _run_item(features, W1, b1, W2, b2, Wc1):
    return pl.pallas_call(
        _item_body,
        grid=(NITEM // BLK,),
        in_specs=[
            pl.BlockSpec((BLK, DF), lambda i: (i, 0)),
            pl.BlockSpec((1024, DF), lambda i: (0, 0)),
            pl.BlockSpec((1024,), lambda i: (0,)),
            pl.BlockSpec((DL, 1024), lambda i: (0, 0)),
            pl.BlockSpec((DL,), lambda i: (0,)),
            pl.BlockSpec((DL, DL), lambda i: (0, 0)),
        ],
        out_specs=[
            pl.BlockSpec((BLK, HALF), lambda i: (i, 0)),
            pl.BlockSpec((BLK, HALF), lambda i: (i, 0)),
        ],
        out_shape=[jax.ShapeDtypeStruct((NITEM, HALF), F32)] * 2,
    )(features, W1, b1, W2, b2, Wc1)


def _run_mid(agglo, agghi, deg, Wc2):
    return pl.pallas_call(
        _mid_body,
        grid=(NPAD // BLK3,),
        in_specs=[
            pl.BlockSpec((BLK3, HALF), lambda i: (i, 0)),
            pl.BlockSpec((BLK3, HALF), lambda i: (i, 0)),
            pl.BlockSpec((BLK3,), lambda i: (i,)),
            pl.BlockSpec((DL, DL), lambda i: (0, 0)),
        ],
        out_specs=[
            pl.BlockSpec((BLK3, HALF), lambda i: (i, 0)),
            pl.BlockSpec((BLK3, HALF), lambda i: (i, 0)),
        ],
        out_shape=[jax.ShapeDtypeStruct((NPAD, HALF), F32)] * 2,
    )(agglo, agghi, deg, Wc2)


def _run_fin(agglo, agghi, deg):
    return pl.pallas_call(
        _fin_body,
        grid=(NPAD // BLK3,),
        in_specs=[
            pl.BlockSpec((BLK3, HALF), lambda i: (i, 0)),
            pl.BlockSpec((BLK3, HALF), lambda i: (i, 0)),
            pl.BlockSpec((BLK3,), lambda i: (i,)),
        ],
        out_specs=pl.BlockSpec((BLK3, DL), lambda i: (i, 0)),
        out_shape=jax.ShapeDtypeStruct((NPAD, DL), F32),
    )(agglo, agghi, deg)


@functools.lru_cache(maxsize=None)
def _make_agg(with_deg):
    mesh = plsc.VectorSubcoreMesh(core_axis_name="c", subcore_axis_name="s")
    out_type = [jax.ShapeDtypeStruct((NPAD, HALF), F32),
                jax.ShapeDtypeStruct((NPAD, HALF), F32)]
    if with_deg:
        out_type.append(jax.ShapeDtypeStruct((NPAD,), F32))
    scratch = [
        pltpu.VMEM_SHARED((NPAD, HALF), F32),
        pltpu.VMEM_SHARED((NPAD,), F32),
        pltpu.VMEM((GROUP, 128), jnp.int32),
        pltpu.VMEM((GROUP, 128), jnp.int32),
        pltpu.VMEM((GROUP, 128, HALF), F32),
        pltpu.VMEM((128,), F32),
        pltpu.VMEM((ZRC, HALF), F32),
        pltpu.VMEM((DEGC,), F32),
        pltpu.SemaphoreType.DMA,
    ]

    def body(tlo, thi, src2, dst2, *rest):
        if with_deg:
            (agglo, agghi, degout, acc, dega, sidx, didx, rows, onesv,
             zrows, zdeg, sem) = rest
        else:
            (agglo, agghi, acc, dega, sidx, didx, rows, onesv,
             zrows, zdeg, sem) = rest
            degout = None
        cid = lax.axis_index("c")
        sid = lax.axis_index("s")
        base = sid * ROWS_PT
        z16 = jnp.zeros((16,), F32)

        def zrow_body(i, carry):
            zrows[i, pl.ds(0, 16)] = z16
            zrows[i, pl.ds(16, 16)] = z16
            return carry
        lax.fori_loop(0, ZRC, zrow_body, 0)

        for k in range(ROWS_PT // ZRC):
            pltpu.sync_copy(zrows, acc.at[pl.ds(base + k * ZRC, ZRC)])

        if with_deg:
            one16 = jnp.full((16,), 1.0, F32)
            for j in range(8):
                onesv[pl.ds(j * 16, 16)] = one16

            def zdeg_body(i, carry):
                zdeg[pl.ds(i * 16, 16)] = z16
                return carry
            lax.fori_loop(0, DEGC // 16, zdeg_body, 0)

            @pl.when(cid == 0)
            def _():
                for k in range(ROWS_PT // DEGC):
                    pltpu.sync_copy(zdeg, dega.at[pl.ds(base + k * DEGC, DEGC)])

        plsc.subcore_barrier()

        def run(table, do_deg):
            def group_body(g, carry):
                crow = sid * CPT + g * GROUP
                pltpu.sync_copy(src2.at[pl.ds(crow, GROUP)], sidx)
                pltpu.sync_copy(dst2.at[pl.ds(crow, GROUP)], didx)
                cps = [pltpu.async_copy(table.at[sidx.at[j]], rows.at[j], sem)
                       for j in range(GROUP)]
                for cp in cps:
                    cp.wait()
                for j in range(GROUP):
                    pltpu.sync_copy(rows.at[j], acc.at[didx.at[j]], add=True)
                    if do_deg:
                        pltpu.sync_copy(onesv, dega.at[didx.at[j]], add=True)
                return carry
            lax.fori_loop(0, CPT // GROUP, group_body, 0)

        @pl.when(cid == 0)
        def _():
            run(tlo, with_deg)

        @pl.when(cid == 1)
        def _():
            run(thi, False)

        plsc.subcore_barrier()

        def drain_to(out_ref):
            for k in range(ROWS_PT // ZRC):
                r0 = base + k * ZRC
                pltpu.sync_copy(acc.at[pl.ds(r0, ZRC)], zrows)
                pltpu.sync_copy(zrows, out_ref.at[pl.ds(r0, ZRC)])

        @pl.when(cid == 0)
        def _():
            drain_to(agglo)

        @pl.when(cid == 1)
        def _():
            drain_to(agghi)

        if with_deg:
            @pl.when(cid == 0)
            def _():
                for k in range(ROWS_PT // DEGC):
                    r0 = base + k * DEGC
                    pltpu.sync_copy(dega.at[pl.ds(r0, DEGC)], zdeg)
                    pltpu.sync_copy(zdeg, degout.at[pl.ds(r0, DEGC)])

    return pl.kernel(body, mesh=mesh, out_type=tuple(out_type),
                     scratch_types=scratch,
                     compiler_params=pltpu.CompilerParams(
                         use_tc_tiling_on_sc=False))


def _dot_body(x2, un, itn, out, uidx, iidx, urows, irows, sbuf, sem):
    cid = lax.axis_index("c")
    sid = lax.axis_index("s")
    w = sid * 2 + cid
    base = w * 32
    pltpu.sync_copy(un.at[pl.ds(base, 32)], uidx)
    pltpu.sync_copy(itn.at[pl.ds(base, 32)], iidx)
    cpu = pltpu.async_copy(x2.at[uidx], urows, sem)
    cpi = pltpu.async_copy(x2.at[iidx], irows, sem)
    cpu.wait()
    cpi.wait()
    lanes = lax.iota(jnp.int32, 16)

    def row_body(r, carry):
        a = urows[r, pl.ds(0, 16)] * irows[r, pl.ds(0, 16)]
        a = a + urows[r, pl.ds(16, 16)] * irows[r, pl.ds(16, 16)]
        a = a + urows[r, pl.ds(32, 16)] * irows[r, pl.ds(32, 16)]
        a = a + urows[r, pl.ds(48, 16)] * irows[r, pl.ds(48, 16)]
        s = jnp.sum(a)
        half = r // 16
        pos = r - half * 16
        off = half * 16
        vec = sbuf[pl.ds(off, 16)]
        sbuf[pl.ds(off, 16)] = jnp.where(lanes == pos, s, vec)
        return carry
    lax.fori_loop(0, 32, row_body, 0)
    pltpu.sync_copy(sbuf, out.at[pl.ds(base, 32)])


@functools.lru_cache(maxsize=None)
def _make_dot():
    return pl.kernel(
        _dot_body,
        mesh=plsc.VectorSubcoreMesh(core_axis_name="c", subcore_axis_name="s"),
        out_type=jax.ShapeDtypeStruct((1024,), F32),
        scratch_types=[
            pltpu.VMEM((32,), jnp.int32),
            pltpu.VMEM((32,), jnp.int32),
            pltpu.VMEM((32, DL), F32),
            pltpu.VMEM((32, DL), F32),
            pltpu.VMEM((32,), F32),
            pltpu.SemaphoreType.DMA,
        ],
        compiler_params=pltpu.CompilerParams(use_tc_tiling_on_sc=False,
                                             needs_layout_passes=False),
    )


def kernel(user_nodes, item_nodes, edge_index, features, preference,
           W1, b1, W2, b2, Wc1, Wc2):
    e = edge_index.astype(jnp.int32)
    src = jnp.concatenate([e[:, 0], e[:, 1]])
    dst = jnp.concatenate([e[:, 1], e[:, 0]])
    npad_e = EPAD - E2
    padidx = NREAL + (jnp.arange(npad_e, dtype=jnp.int32) % 128)
    src2 = jnp.concatenate([src, padidx]).reshape(NCHUNK, 128)
    dst2 = jnp.concatenate([dst, padidx]).reshape(NCHUNK, 128)

    plo, phi = _run_pref(preference, Wc1)
    ilo, ihi = _run_item(features, W1, b1, W2, b2, Wc1)
    zpad = jnp.zeros((NPAD - NREAL, HALF), F32)
    tlo = jnp.concatenate([plo, ilo, zpad], axis=0)
    thi = jnp.concatenate([phi, ihi, zpad], axis=0)

    agg1lo, agg1hi, deg = _make_agg(True)(tlo, thi, src2, dst2)

    xw2lo, xw2hi = _run_mid(agg1lo, agg1hi, deg, Wc2)

    agg2lo, agg2hi = _make_agg(False)(xw2lo, xw2hi, src2, dst2)

    x2 = _run_fin(agg2lo, agg2hi, deg)

    return _make_dot()(x2, user_nodes.astype(jnp.int32),
                       item_nodes.astype(jnp.int32))
